# baseline (speedup 1.0000x reference)
import functools

import jax
import jax.numpy as jnp
from jax import lax
from jax.experimental import pallas as pl
from jax.experimental.pallas import tpu as pltpu
from jax.experimental.pallas import tpu_sc as plsc

N = 8192
D = 256
M = 8192
G = 3
OUTW = 2 * D + 4
TAILW = D + 4

NW = 32
GP_W = M // NW
CH = 32
NCH = GP_W // CH
IDX = CH * G

CT = 5
PADW = CT * 128
TROW = 8 * 128
SLABW = (CH // 8) * CT * TROW


def _tmpl_body(h_ref, o_ref):
    mean = jnp.sum(h_ref[...], axis=0, keepdims=True) * (1.0 / N)
    col = lax.broadcasted_iota(jnp.int32, (1, PADW - 2 * D), 1)
    tail = jnp.where(col == 0, jnp.float32(G / 3.0), jnp.float32(0.0))
    o_ref[...] = jnp.concatenate([mean, tail], axis=1)


def _col_mean_tmpl(h):
    return pl.pallas_call(
        _tmpl_body,
        out_shape=jax.ShapeDtypeStruct((1, PADW - D), jnp.float32),
    )(h)


_mesh = plsc.VectorSubcoreMesh(core_axis_name="c", subcore_axis_name="s")


@functools.partial(
    pl.kernel,
    mesh=_mesh,
    out_type=jax.ShapeDtypeStruct((M // 8 * CT * TROW,), jnp.float32),
    scratch_types=[
        pltpu.VMEM((GP_W * G,), jnp.int32),
        pltpu.VMEM((IDX, D), jnp.float32),
        pltpu.VMEM((IDX, D), jnp.float32),
        pltpu.VMEM((SLABW,), jnp.float32),
        pltpu.VMEM((SLABW,), jnp.float32),
        pltpu.SemaphoreType.DMA,
        pltpu.SemaphoreType.DMA,
        pltpu.SemaphoreType.DMA,
        pltpu.SemaphoreType.DMA,
    ],
)
def _sc_build(h_hbm, gflat_hbm, tmpl_hbm, out_hbm,
              idx_v, rows0, rows1, slab0, slab1,
              sg0, sg1, so0, so1):
    cid = lax.axis_index("c")
    sid = lax.axis_index("s")
    wid = sid * 2 + cid
    base_g = wid * GP_W

    rows = (rows0, rows1)
    slabs = (slab0, slab1)
    gsems = (sg0, sg1)
    osems = (so0, so1)

    pltpu.sync_copy(gflat_hbm.at[pl.ds(base_g * G, GP_W * G)], idx_v)

    for slab_v in slabs:
        for r in range(CH // 8):
            pltpu.sync_copy(
                tmpl_hbm,
                slab_v.at[pl.ds((r * CT + 2) * TROW, 3 * TROW)])

    def start_gather(k):
        b = k % 2
        return pltpu.async_copy(
            h_hbm.at[idx_v.at[pl.ds(k * IDX, IDX)]], rows[b], gsems[b])

    def pool(k):
        b = k % 2
        rows_v, slab_v = rows[b], slabs[b]

        def body_r(r, carry):
            def body_s(s, carry2):
                g = r * 8 + s
                row = g * G
                soff = r * (CT * TROW) + s * 128
                for c in range(16):
                    a = rows_v[row, pl.ds(c * 16, 16)]
                    b2 = rows_v[row + 1, pl.ds(c * 16, 16)]
                    d2 = rows_v[row + 2, pl.ds(c * 16, 16)]
                    off = soff + (c // 8) * TROW + (c % 8) * 16
                    slab_v[pl.ds(off, 16)] = (a + b2 + d2) * jnp.float32(1.0 / G)
                return carry2
            lax.fori_loop(0, 8, body_s, carry)
            return carry

        lax.fori_loop(0, CH // 8, body_r, 0)

    def start_out(k):
        b = k % 2
        tr0 = (base_g + k * CH) // 8
        return pltpu.async_copy(
            slabs[b], out_hbm.at[pl.ds(tr0 * CT * TROW, SLABW)], osems[b])

    ghandles = [None, None]
    ohandles = [None, None]
    ghandles[0] = start_gather(0)
    for k in range(NCH):
        b = k % 2
        if k + 1 < NCH:
            ghandles[1 - b] = start_gather(k + 1)
        ghandles[b].wait()
        if ohandles[b] is not None:
            ohandles[b].wait()
        pool(k)
        ohandles[b] = start_out(k)
    for b in range(2):
        if ohandles[b] is not None:
            ohandles[b].wait()


def kernel(h, groups):
    gflat = groups.astype(jnp.int32).reshape(-1)
    tmpl_row = _col_mean_tmpl(h)
    tmpl = jnp.broadcast_to(tmpl_row.reshape(3, 1, 128), (3, 8, 128)).reshape(-1)
    x_flat = _sc_build(h, gflat, tmpl)
    x4 = x_flat.reshape(M // 8, CT, 8, 128)
    return x4.transpose(0, 2, 1, 3).reshape(M, PADW)[:, :OUTW]


# speedup vs baseline: 1.0515x; 1.0515x over previous
import functools

import jax
import jax.numpy as jnp
from jax import lax
from jax.experimental import pallas as pl
from jax.experimental.pallas import tpu as pltpu
from jax.experimental.pallas import tpu_sc as plsc

N = 8192
D = 256
M = 8192
G = 3
OUTW = 2 * D + 4
TAILW = D + 4

NW = 32
GP_W = M // NW
CH = 32
NCH = GP_W // CH
IDX = CH * G
NGB = 3


def _tmpl_body(h_ref, o_ref):
    mean = jnp.sum(h_ref[...], axis=0, keepdims=True) * (1.0 / N)
    col4 = lax.broadcasted_iota(jnp.int32, (1, 4), 1)
    tail = jnp.where(col4 == 0, jnp.float32(G / 3.0), jnp.float32(0.0))
    o_ref[...] = jnp.concatenate([mean, tail], axis=1)


def _col_mean_tmpl(h):
    return pl.pallas_call(
        _tmpl_body,
        out_shape=jax.ShapeDtypeStruct((1, TAILW), jnp.float32),
    )(h)


_mesh = plsc.VectorSubcoreMesh(core_axis_name="c", subcore_axis_name="s")


@functools.partial(
    pl.kernel,
    mesh=_mesh,
    out_type=jax.ShapeDtypeStruct((M, OUTW), jnp.float32),
    scratch_types=[
        pltpu.VMEM((GP_W * G,), jnp.int32),
        pltpu.VMEM((IDX, D), jnp.float32),
        pltpu.VMEM((IDX, D), jnp.float32),
        pltpu.VMEM((IDX, D), jnp.float32),
        pltpu.VMEM((CH, OUTW), jnp.float32),
        pltpu.VMEM((CH, OUTW), jnp.float32),
        pltpu.SemaphoreType.DMA,
        pltpu.SemaphoreType.DMA,
        pltpu.SemaphoreType.DMA,
        pltpu.SemaphoreType.DMA,
        pltpu.SemaphoreType.DMA,
    ],
)
def _sc_build(h_hbm, gflat_hbm, tmpl_hbm, out_hbm,
              idx_v, rows0, rows1, rows2, slab0, slab1,
              sg0, sg1, sg2, so0, so1):
    cid = lax.axis_index("c")
    sid = lax.axis_index("s")
    wid = sid * 2 + cid
    base_g = wid * GP_W

    rows = (rows0, rows1, rows2)
    slabs = (slab0, slab1)
    gsems = (sg0, sg1, sg2)
    osems = (so0, so1)

    with jax.named_scope("idx_load"):
        pltpu.sync_copy(gflat_hbm.at[pl.ds(base_g * G, GP_W * G)], idx_v)

    def start_gather(k):
        b = k % NGB
        return pltpu.async_copy(
            h_hbm.at[idx_v.at[pl.ds(k * IDX, IDX)]], rows[b], gsems[b])

    ghandles = [None] * NGB
    for k in range(NGB - 1):
        ghandles[k] = start_gather(k)

    with jax.named_scope("tail_fill"):
        for slab_v in slabs:
            pltpu.sync_copy(tmpl_hbm, slab_v.at[:, pl.ds(D, TAILW)])

    def pool(k):
        rows_v, slab_v = rows[k % NGB], slabs[k % 2]

        def body(g, carry):
            r = g * G
            for c in range(16):
                a = rows_v[r, pl.ds(c * 16, 16)]
                b2 = rows_v[r + 1, pl.ds(c * 16, 16)]
                d2 = rows_v[r + 2, pl.ds(c * 16, 16)]
                slab_v[g, pl.ds(c * 16, 16)] = (a + b2 + d2) * jnp.float32(1.0 / G)
            return carry

        lax.fori_loop(0, CH, body, 0)

    def start_out(k):
        g0 = base_g + k * CH
        return pltpu.async_copy(
            slabs[k % 2], out_hbm.at[pl.ds(g0, CH), :], osems[k % 2])

    ohandles = [None, None]
    for k in range(NCH):
        if k + NGB - 1 < NCH:
            ghandles[(k + NGB - 1) % NGB] = start_gather(k + NGB - 1)
        with jax.named_scope("gwait"):
            ghandles[k % NGB].wait()
        if ohandles[k % 2] is not None:
            with jax.named_scope("owait"):
                ohandles[k % 2].wait()
        with jax.named_scope("pool"):
            pool(k)
        ohandles[k % 2] = start_out(k)
    for b in range(2):
        if ohandles[b] is not None:
            ohandles[b].wait()


def kernel(h, groups):
    gflat = groups.astype(jnp.int32).reshape(-1)
    tmpl = jnp.broadcast_to(_col_mean_tmpl(h), (CH, TAILW))
    return _sc_build(h, gflat, tmpl)


# speedup vs baseline: 1.2874x; 1.2243x over previous
import functools

import jax
import jax.numpy as jnp
from jax import lax
from jax.experimental import pallas as pl
from jax.experimental.pallas import tpu as pltpu
from jax.experimental.pallas import tpu_sc as plsc

N = 8192
D = 256
M = 8192
G = 3
OUTW = 2 * D + 4
TAILW = D + 4

NW = 32
GP_W = M // NW
CH = 32
NCH = GP_W // CH
IDX = CH * G
NGB = 3


def _tmpl_body(h_ref, o_ref):
    mean = jnp.sum(h_ref[...], axis=0, keepdims=True) * (1.0 / N)
    col4 = lax.broadcasted_iota(jnp.int32, (1, 4), 1)
    tail = jnp.where(col4 == 0, jnp.float32(G / 3.0), jnp.float32(0.0))
    o_ref[...] = jnp.concatenate([mean, tail], axis=1)


def _col_mean_tmpl(h):
    return pl.pallas_call(
        _tmpl_body,
        out_shape=jax.ShapeDtypeStruct((1, TAILW), jnp.float32),
    )(h)


_mesh = plsc.VectorSubcoreMesh(core_axis_name="c", subcore_axis_name="s")


@functools.partial(
    pl.kernel,
    mesh=_mesh,
    out_type=jax.ShapeDtypeStruct((M, OUTW), jnp.float32),
    scratch_types=[
        pltpu.VMEM((GP_W * G,), jnp.int32),
        pltpu.VMEM((IDX, D), jnp.float32),
        pltpu.VMEM((IDX, D), jnp.float32),
        pltpu.VMEM((IDX, D), jnp.float32),
        pltpu.VMEM((CH, OUTW), jnp.float32),
        pltpu.VMEM((CH, OUTW), jnp.float32),
        pltpu.SemaphoreType.DMA,
        pltpu.SemaphoreType.DMA,
        pltpu.SemaphoreType.DMA,
        pltpu.SemaphoreType.DMA,
        pltpu.SemaphoreType.DMA,
    ],
)
def _sc_build(h_hbm, gflat_hbm, tmpl_hbm, out_hbm,
              idx_v, rows0, rows1, rows2, slab0, slab1,
              sg0, sg1, sg2, so0, so1):
    cid = lax.axis_index("c")
    sid = lax.axis_index("s")
    wid = sid * 2 + cid
    base_g = wid * GP_W

    rows = (rows0, rows1, rows2)
    slabs = (slab0, slab1)
    gsems = (sg0, sg1, sg2)
    osems = (so0, so1)

    with jax.named_scope("idx_load"):
        pltpu.sync_copy(gflat_hbm.at[pl.ds(base_g * G, GP_W * G)], idx_v)

    def start_gather(k):
        b = k % NGB
        return pltpu.async_copy(
            h_hbm.at[idx_v.at[pl.ds(k * IDX, IDX)]], rows[b], gsems[b])

    ghandles = [None] * NGB
    for k in range(NGB - 1):
        ghandles[k] = start_gather(k)

    tail_handles = [
        pltpu.async_copy(tmpl_hbm, slabs[b].at[:, pl.ds(D, TAILW)], osems[b])
        for b in range(2)
    ]

    def pool(k):
        rows_v, slab_v = rows[k % NGB], slabs[k % 2]

        def body(g, carry):
            r = g * G
            for half in range(2):
                cs = [half * 8 + c for c in range(8)]
                a = [rows_v[r, pl.ds(c * 16, 16)] for c in cs]
                b2 = [rows_v[r + 1, pl.ds(c * 16, 16)] for c in cs]
                d2 = [rows_v[r + 2, pl.ds(c * 16, 16)] for c in cs]
                for i, c in enumerate(cs):
                    slab_v[g, pl.ds(c * 16, 16)] = (
                        (a[i] + b2[i] + d2[i]) * jnp.float32(1.0 / G))
            return carry

        lax.fori_loop(0, CH, body, 0)

    def start_out(k):
        g0 = base_g + k * CH
        return pltpu.async_copy(
            slabs[k % 2], out_hbm.at[pl.ds(g0, CH), :], osems[k % 2])

    ohandles = list(tail_handles)
    for k in range(NCH):
        if k + NGB - 1 < NCH:
            ghandles[(k + NGB - 1) % NGB] = start_gather(k + NGB - 1)
        with jax.named_scope("gwait"):
            ghandles[k % NGB].wait()
        if ohandles[k % 2] is not None:
            with jax.named_scope("owait"):
                ohandles[k % 2].wait()
        with jax.named_scope("pool"):
            pool(k)
        ohandles[k % 2] = start_out(k)
    for b in range(2):
        if ohandles[b] is not None:
            ohandles[b].wait()


def kernel(h, groups):
    gflat = groups.astype(jnp.int32).reshape(-1)
    tmpl = jnp.broadcast_to(_col_mean_tmpl(h), (CH, TAILW))
    return _sc_build(h, gflat, tmpl)
